# Initial kernel scaffold; baseline (speedup 1.0000x reference)
import jax
import jax.numpy as jnp
from jax.experimental import pallas as pl


def kernel(x, weight):
    raise NotImplementedError("write your pallas kernel here")


# speedup vs baseline: 1.0220x; 1.0220x over previous
import functools

import jax
import jax.numpy as jnp
from jax import lax
from jax.experimental import pallas as pl
from jax.experimental.pallas import tpu as pltpu
from jax.experimental.pallas import tpu_sc as plsc

B = 16384 * 50
D = 32
NC, NS = 2, 16
NW = NC * NS
CHUNK = 128
CHUNKS_TOTAL = B // CHUNK
CHUNKS_PER_W = CHUNKS_TOTAL // NW

_mesh = plsc.VectorSubcoreMesh(core_axis_name="c", subcore_axis_name="s")


@functools.partial(
    pl.kernel,
    mesh=_mesh,
    out_type=jax.ShapeDtypeStruct((B, D), jnp.float32),
    scratch_types=[
        pltpu.VMEM((CHUNKS_PER_W, CHUNK), jnp.int32),
        pltpu.VMEM((CHUNK, D), jnp.float32),
        pltpu.SemaphoreType.DMA,
    ],
    compiler_params=pltpu.CompilerParams(use_tc_tiling_on_sc=False),
)
def _gather_kernel(idx_hbm, table_hbm, out_hbm, idx_v, rows_v, sem):
    wid = lax.axis_index("c") * NS + lax.axis_index("s")
    base_chunk = wid * CHUNKS_PER_W
    pltpu.sync_copy(idx_hbm.at[pl.ds(base_chunk, CHUNKS_PER_W)], idx_v)

    def body(j, carry):
        pltpu.async_copy(table_hbm.at[idx_v.at[j]], rows_v, sem).wait()
        pltpu.sync_copy(
            rows_v, out_hbm.at[pl.ds((base_chunk + j) * CHUNK, CHUNK)]
        )
        return carry

    lax.fori_loop(0, CHUNKS_PER_W, body, 0)


def kernel(x, weight):
    idx2d = x.reshape(CHUNKS_TOTAL, CHUNK).astype(jnp.int32)
    out = _gather_kernel(idx2d, weight)
    return out.reshape(x.shape + (D,))


# speedup vs baseline: 1.1086x; 1.0847x over previous
import functools

import jax
import jax.numpy as jnp
from jax import lax
from jax.experimental import pallas as pl
from jax.experimental.pallas import tpu as pltpu
from jax.experimental.pallas import tpu_sc as plsc

B = 16384 * 50
D = 32
NC, NS = 2, 16
NW = NC * NS
CHUNK = 128
CHUNKS_TOTAL = B // CHUNK
CHUNKS_PER_W = CHUNKS_TOTAL // NW
K = 10
BLK = K * CHUNK
NBLK = CHUNKS_PER_W // K

_mesh = plsc.VectorSubcoreMesh(core_axis_name="c", subcore_axis_name="s")


@functools.partial(
    pl.kernel,
    mesh=_mesh,
    out_type=jax.ShapeDtypeStruct((B, D), jnp.float32),
    scratch_types=[
        pltpu.VMEM((CHUNKS_PER_W, CHUNK), jnp.int32),
        pltpu.VMEM((2 * BLK, D), jnp.float32),
        pltpu.SemaphoreType.DMA,
        pltpu.SemaphoreType.DMA,
    ],
    compiler_params=pltpu.CompilerParams(use_tc_tiling_on_sc=False),
)
def _gather_kernel(idx_hbm, table_hbm, out_hbm, idx_v, rows_v, gsem, ssem):
    wid = lax.axis_index("c") * NS + lax.axis_index("s")
    base_chunk = wid * CHUNKS_PER_W
    pltpu.sync_copy(idx_hbm.at[pl.ds(base_chunk, CHUNKS_PER_W)], idx_v)

    def drain_store():
        pltpu.make_async_copy(
            out_hbm.at[pl.ds(0, BLK)], rows_v.at[pl.ds(0, BLK)], ssem
        ).wait()

    def body(b, carry):
        off = (b % 2) * BLK
        @pl.when(b >= 2)
        def _():
            drain_store()

        descs = [
            pltpu.async_copy(
                table_hbm.at[idx_v.at[b * K + t]],
                rows_v.at[pl.ds(off + t * CHUNK, CHUNK)],
                gsem,
            )
            for t in range(K)
        ]
        for d in descs:
            d.wait()
        pltpu.async_copy(
            rows_v.at[pl.ds(off, BLK)],
            out_hbm.at[pl.ds((base_chunk + b * K) * CHUNK, BLK)],
            ssem,
        )
        return carry

    lax.fori_loop(0, NBLK, body, 0)
    drain_store()
    drain_store()


def kernel(x, weight):
    idx2d = x.reshape(CHUNKS_TOTAL, CHUNK).astype(jnp.int32)
    out = _gather_kernel(idx2d, weight)
    return out.reshape(x.shape + (D,))
